# baseline (speedup 1.0000x reference)
import jax
import jax.numpy as jnp
from jax import lax
from jax.experimental import pallas as pl
from jax.experimental.pallas import tpu as pltpu
from jax.experimental.pallas import tpu_sc as plsc

BATCH = 16384
HIST = 200
DIM = 32
TOTAL = BATCH * HIST
NC, NS = 2, 16
NW = NC * NS
NB = 4
CHUNK = NB * HIST
BAT_PER_W = BATCH // NW
CHUNKS_PER_W = BAT_PER_W // NB
SPLITS = ((0, 128), (128, 72))


def _gather_body(idx_hbm, table_hbm, out_hbm,
                 idx_v0, idx_v1, rows_v0, rows_v1,
                 si0, si1, sg0, sg1, so0, so1):
    wid = lax.axis_index("s") * NC + lax.axis_index("c")
    bat0 = wid * BAT_PER_W

    idx_v = (idx_v0, idx_v1)
    rows_v = (rows_v0, rows_v1)
    si = (si0, si1)
    sg = (sg0, sg1)
    so = (so0, so1)

    def idx_slice(g):
        return idx_hbm.at[pl.ds((bat0 + g * NB) * HIST, CHUNK)]

    def out_slice(g):
        return out_hbm.at[pl.ds(bat0 + g * NB, NB)]

    def fire_gathers(b):
        for row in range(NB):
            for off, ln in SPLITS:
                pltpu.async_copy(
                    table_hbm.at[idx_v[b].at[pl.ds(row * HIST + off, ln)]],
                    rows_v[b].at[row, pl.ds(off, ln)],
                    sg[b],
                )

    def drain_gathers(b):
        for row in range(NB):
            for off, ln in SPLITS:
                pltpu.make_async_copy(
                    table_hbm.at[idx_v[b].at[pl.ds(row * HIST + off, ln)]],
                    rows_v[b].at[row, pl.ds(off, ln)],
                    sg[b],
                ).wait()

    pltpu.async_copy(idx_slice(0), idx_v[0], si[0])

    def round_fn(r, carry):
        for b in range(2):
            g = r * 2 + b
            ob = 1 - b
            pltpu.make_async_copy(idx_slice(g), idx_v[b], si[b]).wait()

            @pl.when(r > 0)
            def _():
                pltpu.make_async_copy(rows_v[b], out_slice(g), so[b]).wait()

            fire_gathers(b)

            @pl.when(g >= 1)
            def _():
                drain_gathers(ob)
                pltpu.async_copy(rows_v[ob], out_slice(g - 1), so[ob])

            @pl.when(g + 1 < CHUNKS_PER_W)
            def _():
                pltpu.async_copy(idx_slice(g + 1), idx_v[ob], si[ob])
        return carry

    lax.fori_loop(0, CHUNKS_PER_W // 2, round_fn, 0)

    last = CHUNKS_PER_W - 1
    drain_gathers(1)
    pltpu.async_copy(rows_v[1], out_slice(last), so[1])
    pltpu.make_async_copy(rows_v[0], out_slice(last - 1), so[0]).wait()
    pltpu.make_async_copy(rows_v[1], out_slice(last), so[1]).wait()


def kernel(input, weight):
    idx1d = input.reshape(TOTAL).astype(jnp.int32)
    mesh = plsc.VectorSubcoreMesh(core_axis_name="c", subcore_axis_name="s")
    return pl.kernel(
        _gather_body,
        mesh=mesh,
        out_type=jax.ShapeDtypeStruct((BATCH, HIST, DIM), jnp.float32),
        scratch_types=[
            pltpu.VMEM((CHUNK,), jnp.int32),
            pltpu.VMEM((CHUNK,), jnp.int32),
            pltpu.VMEM((NB, HIST, DIM), jnp.float32),
            pltpu.VMEM((NB, HIST, DIM), jnp.float32),
            pltpu.SemaphoreType.DMA,
            pltpu.SemaphoreType.DMA,
            pltpu.SemaphoreType.DMA,
            pltpu.SemaphoreType.DMA,
            pltpu.SemaphoreType.DMA,
            pltpu.SemaphoreType.DMA,
        ],
        compiler_params=pltpu.CompilerParams(use_tc_tiling_on_sc=False),
    )(idx1d, weight)
